# baseline (speedup 1.0000x reference)
import numpy as np
import jax
import jax.numpy as jnp
from jax import lax
from jax.experimental import pallas as pl
from jax.experimental.pallas import tpu as pltpu
from jax.experimental.pallas import tpu_sc as plsc

_B, _N = 4096, 128
_M = _N * _N
_K = _N * (_N + 1) // 2
_TK = _K // 8
_NW = 32
_BPW = _B // _NW

_ROW_IDX, _COL_IDX = np.triu_indices(_N)
_FLAT_IDX = (_ROW_IDX * _N + _COL_IDX).astype(np.int32)
_SEG_OFF = np.concatenate([[0], np.cumsum(np.arange(_N, 0, -1))])

_WIN = 256
_NWIN = (_K + _WIN - 1) // _WIN

_WPLAN = []
for _w in range(_NWIN):
    _k0 = _w * _WIN
    _k1 = min(_k0 + _WIN, _K)
    _i0 = int(_ROW_IDX[_k0])
    _i1 = int(_ROW_IDX[_k1 - 1]) + 1
    steps = []
    for _ks in range(_k0, _k1, 16):
        _src = None
        if _ROW_IDX[_ks] == _ROW_IDX[_ks + 15]:
            _src = int(_FLAT_IDX[_ks]) - _i0 * _N
        steps.append((_ks, _src))
    _WPLAN.append((_k0, _i0, (_i1 - _i0) * _N, steps))

_SLAB_MAX = max(_p[2] for _p in _WPLAN)


def _sc_body(x_hbm, fidx_hbm, out_hbm, fidx_v, slab0, slab1, area,
             isem, osem):
    c = lax.axis_index("c")
    s = lax.axis_index("s")
    wkr = s * 2 + c
    bg0 = wkr * _BPW

    pltpu.sync_copy(fidx_hbm, fidx_v)

    iota = lax.iota(jnp.int32, 16)
    iota_d8 = lax.shift_right_logical(iota, 3)
    iota_m8 = lax.bitwise_and(iota, 7)
    slabs = (slab0, slab1)

    def start_in(p, wlen, base, b):
        pltpu.async_copy(x_hbm.at[bg0 + b, pl.ds(base, wlen)],
                         slabs[p].at[pl.ds(0, wlen)], isem)

    def wait_in(p, wlen, base, b):
        pltpu.make_async_copy(x_hbm.at[bg0 + b, pl.ds(base, wlen)],
                              slabs[p].at[pl.ds(0, wlen)], isem).wait()

    for w, (k0, i0, wlen, steps) in enumerate(_WPLAN):
        nt = len(steps) * 2
        ap = w & 1
        base = i0 * _N

        if w >= 2:
            nt_prev = len(_WPLAN[w - 2][3]) * 2
            pltpu.make_async_copy(
                area.at[w & 1, pl.ds(0, nt_prev)],
                out_hbm.at[pl.ds((w - 2) * (_WIN // 8), nt_prev), wkr],
                osem).wait()

        if w == 0:
            start_in(0, wlen, base, 0)
            start_in(1, wlen, base, 1)

        def pairs(bb, carry):
            for p in (0, 1):
                b = bb * 2 + p
                wait_in(p, wlen, base, b)
                bvec = jnp.full((16,), b, dtype=jnp.int32)
                vals = []
                for ks, src in steps:
                    if src is not None:
                        vals.append(slabs[p][pl.ds(src, 16)])
                    else:
                        fv = fidx_v[pl.ds(ks, 16)] - base
                        vals.append(plsc.load_gather(slabs[p], [fv]))

                @pl.when(b + 2 < _BPW)
                def _():
                    start_in(p, wlen, base, b + 2)

                for (ks, src), v in zip(steps, vals):
                    kloc = ks - k0
                    tv = iota_d8 + (kloc // 8)
                    plsc.store_scatter(area.at[ap], [tv, iota_m8, bvec], v)
            return carry

        lax.fori_loop(0, _BPW // 2, pairs, 0)

        pltpu.async_copy(area.at[ap, pl.ds(0, nt)],
                         out_hbm.at[pl.ds(w * (_WIN // 8), nt), wkr], osem)

        if w + 1 < _NWIN:
            nk0, ni0, nwlen, _ = _WPLAN[w + 1]
            start_in(0, nwlen, ni0 * _N, 0)
            start_in(1, nwlen, ni0 * _N, 1)

    for w in (_NWIN - 2, _NWIN - 1):
        nt = len(_WPLAN[w][3]) * 2
        pltpu.make_async_copy(
            area.at[w & 1, pl.ds(0, nt)],
            out_hbm.at[pl.ds(w * (_WIN // 8), nt), wkr], osem).wait()


def kernel(input):
    x2 = input.reshape(_B, _M)
    fidx = jnp.asarray(_FLAT_IDX)
    mesh = plsc.VectorSubcoreMesh(core_axis_name="c", subcore_axis_name="s")
    f = pl.kernel(
        _sc_body,
        mesh=mesh,
        out_type=jax.ShapeDtypeStruct((_TK, _NW, 8, _BPW), jnp.float32),
        scratch_types=[
            pltpu.VMEM((_K,), jnp.int32),
            pltpu.VMEM((_SLAB_MAX,), jnp.float32),
            pltpu.VMEM((_SLAB_MAX,), jnp.float32),
            pltpu.VMEM((2, _WIN // 8, 8, _BPW), jnp.float32),
            pltpu.SemaphoreType.DMA,
            pltpu.SemaphoreType.DMA,
        ],
        compiler_params=pltpu.CompilerParams(
            use_tc_tiling_on_sc=False, needs_layout_passes=False
        ),
    )
    r4 = f(x2, fidx)
    return r4.transpose(1, 3, 0, 2).reshape(_B, _K)


# speedup vs baseline: 3.0965x; 3.0965x over previous
import numpy as np
import jax
import jax.numpy as jnp
from jax import lax
from jax.experimental import pallas as pl
from jax.experimental.pallas import tpu as pltpu
from jax.experimental.pallas import tpu_sc as plsc

_B, _N = 4096, 128
_M = _N * _N
_K = _N * (_N + 1) // 2
_NT = _K // 16

_NW = 32
_BPW = _B // _NW

_ROW_IDX, _COL_IDX = np.triu_indices(_N)
_FLAT_IDX = (_ROW_IDX * _N + _COL_IDX).astype(np.int32)

_SEG_OFF = np.concatenate([[0], np.cumsum(np.arange(_N, 0, -1))])
_TILE_PLAN = []
for _t in range(_NT):
    _lo = 16 * _t
    _i = int(np.searchsorted(_SEG_OFF, _lo, side="right") - 1)
    if _SEG_OFF[_i + 1] >= _lo + 16:
        _TILE_PLAN.append((_lo, _i * (_N + 1) + (_lo - int(_SEG_OFF[_i]))))
    else:
        _TILE_PLAN.append((_lo, None))


def _sc_body(x_hbm, idx_hbm, out_hbm, idx_v, in0, in1, ou0, ou1,
             is0, is1, os0, os1):
    c = lax.axis_index("c")
    s = lax.axis_index("s")
    wid = s * 2 + c
    b0 = wid * _BPW

    pltpu.sync_copy(idx_hbm, idx_v)

    bufs = ((in0, ou0, is0, os0), (in1, ou1, is1, os1))

    def start_in(p, b):
        iv, _, isem, _ = bufs[p]
        pltpu.async_copy(x_hbm.at[b], iv, isem)

    def wait_in(p, b):
        iv, _, isem, _ = bufs[p]
        pltpu.make_async_copy(x_hbm.at[b], iv, isem).wait()

    def start_out(p, b):
        _, ov, _, osem = bufs[p]
        pltpu.async_copy(ov, out_hbm.at[b], osem)

    def wait_out(p, b):
        _, ov, _, osem = bufs[p]
        pltpu.make_async_copy(ov, out_hbm.at[b], osem).wait()

    start_in(0, b0)
    start_in(1, b0 + 1)

    def pair(rr, carry):
        for p in (0, 1):
            r = rr * 2 + p
            b = b0 + r
            iv, ov, _, _ = bufs[p]
            wait_in(p, b)

            @pl.when(rr > 0)
            def _():
                wait_out(p, b - 2)

            for o, so in _TILE_PLAN:
                if so is not None:
                    ov[pl.ds(o, 16)] = iv[pl.ds(so, 16)]
                else:
                    idx = idx_v[pl.ds(o, 16)]
                    ov[pl.ds(o, 16)] = plsc.load_gather(iv, [idx])
            start_out(p, b)

            @pl.when(r + 2 < _BPW)
            def _():
                start_in(p, b + 2)
        return carry

    lax.fori_loop(0, _BPW // 2, pair, 0)

    wait_out(0, b0 + _BPW - 2)
    wait_out(1, b0 + _BPW - 1)


def kernel(input):
    x2 = input.reshape(_B, _M)
    fidx = jnp.asarray(_FLAT_IDX)
    mesh = plsc.VectorSubcoreMesh(core_axis_name="c", subcore_axis_name="s")
    f = pl.kernel(
        _sc_body,
        mesh=mesh,
        out_type=jax.ShapeDtypeStruct((_B, _K), jnp.float32),
        scratch_types=[
            pltpu.VMEM((_K,), jnp.int32),
            pltpu.VMEM((_M,), jnp.float32),
            pltpu.VMEM((_M,), jnp.float32),
            pltpu.VMEM((_K,), jnp.float32),
            pltpu.VMEM((_K,), jnp.float32),
            pltpu.SemaphoreType.DMA,
            pltpu.SemaphoreType.DMA,
            pltpu.SemaphoreType.DMA,
            pltpu.SemaphoreType.DMA,
        ],
        compiler_params=pltpu.CompilerParams(
            use_tc_tiling_on_sc=False, needs_layout_passes=False
        ),
    )
    return f(x2, fidx)
